# baseline (speedup 1.0000x reference)
import functools

import jax
import jax.numpy as jnp
from jax import lax
from jax.experimental import pallas as pl
from jax.experimental.pallas import tpu as pltpu
from jax.experimental.pallas import tpu_sc as plsc

B = 8
N = 4096
S = 1024
K = 32
M = B * S * K
EPS = 1e-5
BIG = 2 ** 30
INF = float("inf")


def _fps_body(xyzT_ref, far0_ref, cent_ref):
    X3 = xyzT_ref[...]
    iota_n = lax.broadcasted_iota(jnp.int32, (B, N), 1)
    iota_s = lax.broadcasted_iota(jnp.int32, (B, S), 1)

    D0 = jnp.full((B, N), 1e10, dtype=jnp.float32)
    far0 = far0_ref[:, 0:1]
    C0 = jnp.zeros((B, S), dtype=jnp.float32)

    def body(t, carry):
        D, far, Ca, Cb, Cc = carry
        oh = iota_n == far
        cent = jnp.sum(jnp.where(oh[None], X3, 0.0), axis=2, keepdims=True)
        sel = iota_s == t
        Ca = jnp.where(sel, cent[0], Ca)
        Cb = jnp.where(sel, cent[1], Cb)
        Cc = jnp.where(sel, cent[2], Cc)
        d = jnp.sum((X3 - cent) ** 2, axis=0)
        D = jnp.minimum(D, d)
        mx = jnp.max(D, axis=1, keepdims=True)
        far = jnp.min(jnp.where(D == mx, iota_n, BIG), axis=1, keepdims=True)
        return D, far, Ca, Cb, Cc

    _, _, Ca, Cb, Cc = lax.fori_loop(0, S, body, (D0, far0, C0, C0, C0))
    cent_ref[0] = Ca
    cent_ref[1] = Cb
    cent_ref[2] = Cc


def _run_fps(xyzT3, far0):
    return pl.pallas_call(
        _fps_body,
        out_shape=jax.ShapeDtypeStruct((3, B, S), jnp.float32),
    )(xyzT3, far0)


TK_ROWS = 256


def _topk_body(c_ref, xyzT_ref, out_ref):
    b = pl.program_id(0) // (S // TK_ROWS)
    P = xyzT_ref[0]
    C = c_ref[...]
    d = ((C[:, 0:1] - P[0:1, :]) ** 2
         + (C[:, 1:2] - P[1:2, :]) ** 2
         + (C[:, 2:3] - P[2:3, :]) ** 2)
    iota = lax.broadcasted_iota(jnp.int32, (TK_ROWS, N), 1)
    off = (b * N).astype(jnp.int32)
    E = 4
    for r in range(K // E):
        ixs = []
        em = None
        for e in range(E):
            if em is None:
                mn = jnp.min(d, axis=1, keepdims=True)
                cand = jnp.where(d == mn, iota, BIG)
            else:
                mn = jnp.min(jnp.where(em, INF, d), axis=1, keepdims=True)
                cand = jnp.where((d == mn) & (~em), iota, BIG)
            ix = jnp.min(cand, axis=1, keepdims=True)
            out_ref[:, r * E + e:r * E + e + 1] = ix + off
            ixs.append(ix)
            m = iota == ix
            em = m if em is None else em | m
        d = jnp.where(em, INF, d)


def _run_topk(new_xyz_flat, xyzT):
    grid = (B * S // TK_ROWS,)
    return pl.pallas_call(
        _topk_body,
        grid=grid,
        in_specs=[
            pl.BlockSpec((TK_ROWS, 3), lambda i: (i, 0)),
            pl.BlockSpec((1, 3, N), lambda i: (i // (S // TK_ROWS), 0, 0)),
        ],
        out_specs=pl.BlockSpec((TK_ROWS, K), lambda i: (i, 0)),
        out_shape=jax.ShapeDtypeStruct((B * S, K), jnp.int32),
    )(new_xyz_flat, xyzT)


NW = 32
ROWS_PER_W = M // NW
CHUNKS = ROWS_PER_W // 128


@functools.lru_cache(maxsize=1)
def _make_sc_gather():
    mesh = plsc.VectorSubcoreMesh(core_axis_name="c", subcore_axis_name="s")

    @functools.partial(
        pl.kernel,
        mesh=mesh,
        out_type=jax.ShapeDtypeStruct((NW, CHUNKS, 128, 16), jnp.float32),
        scratch_types=[
            pltpu.VMEM((CHUNKS, 128), jnp.int32),
            pltpu.VMEM((128, 16), jnp.float32),
            pltpu.SemaphoreType.DMA,
        ],
        compiler_params=pltpu.CompilerParams(use_tc_tiling_on_sc=False),
    )
    def gather_k(table_hbm, idx_hbm, out_hbm, idx_v, rows_v, sem):
        wid = lax.axis_index("s") * 2 + lax.axis_index("c")
        pltpu.sync_copy(idx_hbm.at[wid], idx_v)

        def body(j, carry):
            pltpu.async_copy(table_hbm.at[idx_v.at[j]], rows_v, sem).wait()
            pltpu.sync_copy(rows_v, out_hbm.at[wid, j])
            return carry

        lax.fori_loop(0, CHUNKS, body, 0, unroll=False)

    return gather_k


def _sc_gather(table, idx_sc):
    return _make_sc_gather()(table, idx_sc)


TILE = 4096
GROUPS = TILE // K
GRID_M = M // TILE


def _stage1_pre(x0b, centb, W1p, b1):
    u = jnp.dot(x0b, W1p.T, preferred_element_type=jnp.float32)
    corr = jnp.dot(centb, W1p[:, 0:3].T, preferred_element_type=jnp.float32)
    corr = jnp.broadcast_to(corr[:, None, :], (GROUPS, K, 64)).reshape(TILE, 64)
    return u - corr + b1


def _bn_coefs(st_ref, g, bt, width):
    s = st_ref[0:1, 0:width]
    q = st_ref[1:2, 0:width]
    mean = s * (1.0 / M)
    var = q * (1.0 / M) - mean * mean
    a = g * lax.rsqrt(var + EPS)
    c = bt - mean * a
    return a, c


def _accum_stats(st_ref, ypre, width):
    @pl.when(pl.program_id(0) == 0)
    def _():
        st_ref[...] = jnp.zeros_like(st_ref)
    st_ref[0:1, 0:width] += jnp.sum(ypre, axis=0, keepdims=True)
    st_ref[1:2, 0:width] += jnp.sum(ypre * ypre, axis=0, keepdims=True)


def _m1_body(x0_ref, cent_ref, W1p_ref, PA_ref, st1_ref):
    b1 = PA_ref[0:1, 0:64]
    ypre = _stage1_pre(x0_ref[...], cent_ref[...], W1p_ref[...], b1)
    _accum_stats(st1_ref, ypre, 64)


def _m2_body(x0_ref, cent_ref, W1p_ref, W2_ref, PA_ref, st1_ref, st2_ref):
    b1 = PA_ref[0:1, 0:64]
    y1pre = _stage1_pre(x0_ref[...], cent_ref[...], W1p_ref[...], b1)
    a1, c1 = _bn_coefs(st1_ref, PA_ref[1:2, 0:64], PA_ref[2:3, 0:64], 64)
    y1 = jnp.maximum(y1pre * a1 + c1, 0.0)
    y2pre = jnp.dot(y1, W2_ref[...].T, preferred_element_type=jnp.float32) \
        + PA_ref[3:4, 0:64]
    _accum_stats(st2_ref, y2pre, 64)


def _m3_body(x0_ref, cent_ref, W1p_ref, W2_ref, W3_ref, PA_ref, PB_ref,
             st1_ref, st2_ref, st3_ref):
    b1 = PA_ref[0:1, 0:64]
    y1pre = _stage1_pre(x0_ref[...], cent_ref[...], W1p_ref[...], b1)
    a1, c1 = _bn_coefs(st1_ref, PA_ref[1:2, 0:64], PA_ref[2:3, 0:64], 64)
    y1 = jnp.maximum(y1pre * a1 + c1, 0.0)
    y2pre = jnp.dot(y1, W2_ref[...].T, preferred_element_type=jnp.float32) \
        + PA_ref[3:4, 0:64]
    a2, c2 = _bn_coefs(st2_ref, PA_ref[4:5, 0:64], PA_ref[5:6, 0:64], 64)
    y2 = jnp.maximum(y2pre * a2 + c2, 0.0)
    y3pre = jnp.dot(y2, W3_ref[...].T, preferred_element_type=jnp.float32) \
        + PB_ref[0:1, :]
    _accum_stats(st3_ref, y3pre, 128)


def _m4_body(x0_ref, cent_ref, W1p_ref, W2_ref, W3_ref, PA_ref, PB_ref,
             st1_ref, st2_ref, st3_ref, out_ref):
    b1 = PA_ref[0:1, 0:64]
    y1pre = _stage1_pre(x0_ref[...], cent_ref[...], W1p_ref[...], b1)
    a1, c1 = _bn_coefs(st1_ref, PA_ref[1:2, 0:64], PA_ref[2:3, 0:64], 64)
    y1 = jnp.maximum(y1pre * a1 + c1, 0.0)
    y2pre = jnp.dot(y1, W2_ref[...].T, preferred_element_type=jnp.float32) \
        + PA_ref[3:4, 0:64]
    a2, c2 = _bn_coefs(st2_ref, PA_ref[4:5, 0:64], PA_ref[5:6, 0:64], 64)
    y2 = jnp.maximum(y2pre * a2 + c2, 0.0)
    y3pre = jnp.dot(y2, W3_ref[...].T, preferred_element_type=jnp.float32) \
        + PB_ref[0:1, :]
    a3, c3 = _bn_coefs(st3_ref, PB_ref[1:2, :], PB_ref[2:3, :], 128)
    y3 = jnp.maximum(y3pre * a3 + c3, 0.0)
    out_ref[...] = jnp.max(y3.reshape(GROUPS, K, 128), axis=1)


def _whole(shape):
    nd = len(shape)
    return pl.BlockSpec(shape, lambda i: (0,) * nd)


def _x0_spec():
    return pl.BlockSpec((TILE, 16), lambda i: (i, 0))


def _cent_spec():
    return pl.BlockSpec((GROUPS, 3), lambda i: (i, 0))


_ST = jax.ShapeDtypeStruct((8, 128), jnp.float32)
_ST_SPEC = pl.BlockSpec((8, 128), lambda i: (0, 0))


def _run_m1(x0, cent, W1p, PA):
    return pl.pallas_call(
        _m1_body, grid=(GRID_M,),
        in_specs=[_x0_spec(), _cent_spec(), _whole((64, 16)), _whole((8, 128))],
        out_specs=_ST_SPEC, out_shape=_ST,
    )(x0, cent, W1p, PA)


def _run_m2(x0, cent, W1p, W2, PA, st1):
    return pl.pallas_call(
        _m2_body, grid=(GRID_M,),
        in_specs=[_x0_spec(), _cent_spec(), _whole((64, 16)), _whole((64, 64)),
                  _whole((8, 128)), _whole((8, 128))],
        out_specs=_ST_SPEC, out_shape=_ST,
    )(x0, cent, W1p, W2, PA, st1)


def _run_m3(x0, cent, W1p, W2, W3, PA, PB, st1, st2):
    return pl.pallas_call(
        _m3_body, grid=(GRID_M,),
        in_specs=[_x0_spec(), _cent_spec(), _whole((64, 16)), _whole((64, 64)),
                  _whole((128, 64)), _whole((8, 128)), _whole((8, 128)),
                  _whole((8, 128)), _whole((8, 128))],
        out_specs=_ST_SPEC, out_shape=_ST,
    )(x0, cent, W1p, W2, W3, PA, PB, st1, st2)


def _run_m4(x0, cent, W1p, W2, W3, PA, PB, st1, st2, st3):
    return pl.pallas_call(
        _m4_body, grid=(GRID_M,),
        in_specs=[_x0_spec(), _cent_spec(), _whole((64, 16)), _whole((64, 64)),
                  _whole((128, 64)), _whole((8, 128)), _whole((8, 128)),
                  _whole((8, 128)), _whole((8, 128)), _whole((8, 128))],
        out_specs=pl.BlockSpec((GROUPS, 128), lambda i: (i, 0)),
        out_shape=jax.ShapeDtypeStruct((B * S, 128), jnp.float32),
    )(x0, cent, W1p, W2, W3, PA, PB, st1, st2, st3)


@jax.jit
def kernel(xyz, points, W1, b1, g1, bt1, W2, b2, g2, bt2, W3, b3, g3, bt3):
    xyzT = jnp.transpose(xyz, (0, 2, 1))
    xyzT3 = jnp.transpose(xyz, (2, 0, 1))
    far0 = jax.random.randint(jax.random.key(42), (B,), 0, N)
    far0 = jnp.broadcast_to(far0.astype(jnp.int32)[:, None], (B, 128))

    cents = _run_fps(xyzT3, far0)
    new_xyz = jnp.transpose(cents, (1, 2, 0))
    new_xyz_flat = new_xyz.reshape(B * S, 3)

    gidx = _run_topk(new_xyz_flat, xyzT)
    idx_sc = gidx.reshape(NW, CHUNKS, 128)

    table = jnp.concatenate(
        [xyz, points, jnp.zeros((B, N, 7), jnp.float32)], axis=-1
    ).reshape(B * N, 16)

    x0 = _sc_gather(table, idx_sc).reshape(M, 16)

    W1p = jnp.concatenate([W1, jnp.zeros((64, 7), jnp.float32)], axis=1)
    pad64 = lambda v: jnp.concatenate([v, jnp.zeros((64,), jnp.float32)])
    PA = jnp.stack([pad64(b1), pad64(g1), pad64(bt1),
                    pad64(b2), pad64(g2), pad64(bt2),
                    jnp.zeros((128,), jnp.float32),
                    jnp.zeros((128,), jnp.float32)])
    PB = jnp.stack([b3, g3, bt3] + [jnp.zeros((128,), jnp.float32)] * 5)

    st1 = _run_m1(x0, new_xyz_flat, W1p, PA)
    st2 = _run_m2(x0, new_xyz_flat, W1p, W2, PA, st1)
    st3 = _run_m3(x0, new_xyz_flat, W1p, W2, W3, PA, PB, st1, st2)
    pooled = _run_m4(x0, new_xyz_flat, W1p, W2, W3, PA, PB, st1, st2, st3)

    return new_xyz, pooled.reshape(B, S, 128)


# speedup vs baseline: 1.5006x; 1.5006x over previous
import functools

import jax
import jax.numpy as jnp
from jax import lax
from jax.experimental import pallas as pl
from jax.experimental.pallas import tpu as pltpu
from jax.experimental.pallas import tpu_sc as plsc

B = 8
N = 4096
S = 1024
K = 32
M = B * S * K
EPS = 1e-5
BIG = 2 ** 30
INF = float("inf")


def _fps_body(xyzT_ref, far0_ref, cent_ref):
    X3 = xyzT_ref[...]
    iota_n = lax.broadcasted_iota(jnp.int32, (B, N), 1)

    D0 = jnp.full((B, N), 1e10, dtype=jnp.float32)
    far0 = far0_ref[:, 0:1]

    def body(t, carry):
        D, far = carry
        oh = iota_n == far
        cent = jnp.sum(jnp.where(oh[None], X3, 0.0), axis=2)
        cent_ref[pl.ds(t, 1), :, :] = cent.reshape(1, 3, B)
        d = jnp.sum((X3 - cent[:, :, None]) ** 2, axis=0)
        D = jnp.minimum(D, d)
        mx = jnp.max(D, axis=1, keepdims=True)
        far = jnp.min(jnp.where(D == mx, iota_n, BIG), axis=1, keepdims=True)
        return D, far

    lax.fori_loop(0, S, body, (D0, far0))


def _run_fps(xyzT3, far0):
    return pl.pallas_call(
        _fps_body,
        out_shape=jax.ShapeDtypeStruct((S, 3, B), jnp.float32),
    )(xyzT3, far0)


TK_ROWS = 256


def _topk_body(c_ref, xyzT_ref, out_ref):
    b = pl.program_id(0) // (S // TK_ROWS)
    P = xyzT_ref[0]
    C = c_ref[...]
    d = ((C[:, 0:1] - P[0:1, :]) ** 2
         + (C[:, 1:2] - P[1:2, :]) ** 2
         + (C[:, 2:3] - P[2:3, :]) ** 2)
    iota = lax.broadcasted_iota(jnp.int32, (TK_ROWS, N), 1)
    off = (b * N).astype(jnp.int32)
    for j in range(K):
        mn = jnp.min(d, axis=1, keepdims=True)
        ix = jnp.min(jnp.where(d == mn, iota, BIG), axis=1, keepdims=True)
        out_ref[:, j:j + 1] = ix + off
        d = jnp.where(iota == ix, INF, d)


def _run_topk(new_xyz_flat, xyzT):
    grid = (B * S // TK_ROWS,)
    return pl.pallas_call(
        _topk_body,
        grid=grid,
        in_specs=[
            pl.BlockSpec((TK_ROWS, 3), lambda i: (i, 0)),
            pl.BlockSpec((1, 3, N), lambda i: (i // (S // TK_ROWS), 0, 0)),
        ],
        out_specs=pl.BlockSpec((TK_ROWS, K), lambda i: (i, 0)),
        out_shape=jax.ShapeDtypeStruct((B * S, K), jnp.int32),
    )(new_xyz_flat, xyzT)


NW = 32
ROWS_PER_W = M // NW
CHUNKS = ROWS_PER_W // 128


@functools.lru_cache(maxsize=1)
def _make_sc_gather():
    mesh = plsc.VectorSubcoreMesh(core_axis_name="c", subcore_axis_name="s")

    @functools.partial(
        pl.kernel,
        mesh=mesh,
        out_type=jax.ShapeDtypeStruct((NW, CHUNKS, 128, 16), jnp.float32),
        scratch_types=[
            pltpu.VMEM((CHUNKS, 128), jnp.int32),
            pltpu.VMEM((128, 16), jnp.float32),
            pltpu.SemaphoreType.DMA,
        ],
        compiler_params=pltpu.CompilerParams(use_tc_tiling_on_sc=False),
    )
    def gather_k(table_hbm, idx_hbm, out_hbm, idx_v, rows_v, sem):
        wid = lax.axis_index("s") * 2 + lax.axis_index("c")
        pltpu.sync_copy(idx_hbm.at[wid], idx_v)

        def body(j, carry):
            pltpu.async_copy(table_hbm.at[idx_v.at[j]], rows_v, sem).wait()
            pltpu.sync_copy(rows_v, out_hbm.at[wid, j])
            return carry

        lax.fori_loop(0, CHUNKS, body, 0, unroll=False)

    return gather_k


def _sc_gather(table, idx_sc):
    return _make_sc_gather()(table, idx_sc)


TILE = 4096
GROUPS = TILE // K
GRID_M = M // TILE


def _stage1_pre(x0b, centb, W1p, b1):
    u = jnp.dot(x0b, W1p.T, preferred_element_type=jnp.float32)
    corr = jnp.dot(centb, W1p[:, 0:3].T, preferred_element_type=jnp.float32)
    corr = jnp.broadcast_to(corr[:, None, :], (GROUPS, K, 64)).reshape(TILE, 64)
    return u - corr + b1


def _bn_coefs(st_ref, g, bt, width):
    s = st_ref[0:1, 0:width]
    q = st_ref[1:2, 0:width]
    mean = s * (1.0 / M)
    var = q * (1.0 / M) - mean * mean
    a = g * lax.rsqrt(var + EPS)
    c = bt - mean * a
    return a, c


def _accum_stats(st_ref, ypre, width):
    @pl.when(pl.program_id(0) == 0)
    def _():
        st_ref[...] = jnp.zeros_like(st_ref)
    st_ref[0:1, 0:width] += jnp.sum(ypre, axis=0, keepdims=True)
    st_ref[1:2, 0:width] += jnp.sum(ypre * ypre, axis=0, keepdims=True)


def _m1_body(x0_ref, cent_ref, W1p_ref, PA_ref, st1_ref):
    b1 = PA_ref[0:1, 0:64]
    ypre = _stage1_pre(x0_ref[...], cent_ref[...], W1p_ref[...], b1)
    _accum_stats(st1_ref, ypre, 64)


def _m2_body(x0_ref, cent_ref, W1p_ref, W2_ref, PA_ref, st1_ref, st2_ref):
    b1 = PA_ref[0:1, 0:64]
    y1pre = _stage1_pre(x0_ref[...], cent_ref[...], W1p_ref[...], b1)
    a1, c1 = _bn_coefs(st1_ref, PA_ref[1:2, 0:64], PA_ref[2:3, 0:64], 64)
    y1 = jnp.maximum(y1pre * a1 + c1, 0.0)
    y2pre = jnp.dot(y1, W2_ref[...].T, preferred_element_type=jnp.float32) \
        + PA_ref[3:4, 0:64]
    _accum_stats(st2_ref, y2pre, 64)


def _m3_body(x0_ref, cent_ref, W1p_ref, W2_ref, W3_ref, PA_ref, PB_ref,
             st1_ref, st2_ref, st3_ref):
    b1 = PA_ref[0:1, 0:64]
    y1pre = _stage1_pre(x0_ref[...], cent_ref[...], W1p_ref[...], b1)
    a1, c1 = _bn_coefs(st1_ref, PA_ref[1:2, 0:64], PA_ref[2:3, 0:64], 64)
    y1 = jnp.maximum(y1pre * a1 + c1, 0.0)
    y2pre = jnp.dot(y1, W2_ref[...].T, preferred_element_type=jnp.float32) \
        + PA_ref[3:4, 0:64]
    a2, c2 = _bn_coefs(st2_ref, PA_ref[4:5, 0:64], PA_ref[5:6, 0:64], 64)
    y2 = jnp.maximum(y2pre * a2 + c2, 0.0)
    y3pre = jnp.dot(y2, W3_ref[...].T, preferred_element_type=jnp.float32) \
        + PB_ref[0:1, :]
    _accum_stats(st3_ref, y3pre, 128)


def _m4_body(x0_ref, cent_ref, W1p_ref, W2_ref, W3_ref, PA_ref, PB_ref,
             st1_ref, st2_ref, st3_ref, out_ref):
    b1 = PA_ref[0:1, 0:64]
    y1pre = _stage1_pre(x0_ref[...], cent_ref[...], W1p_ref[...], b1)
    a1, c1 = _bn_coefs(st1_ref, PA_ref[1:2, 0:64], PA_ref[2:3, 0:64], 64)
    y1 = jnp.maximum(y1pre * a1 + c1, 0.0)
    y2pre = jnp.dot(y1, W2_ref[...].T, preferred_element_type=jnp.float32) \
        + PA_ref[3:4, 0:64]
    a2, c2 = _bn_coefs(st2_ref, PA_ref[4:5, 0:64], PA_ref[5:6, 0:64], 64)
    y2 = jnp.maximum(y2pre * a2 + c2, 0.0)
    y3pre = jnp.dot(y2, W3_ref[...].T, preferred_element_type=jnp.float32) \
        + PB_ref[0:1, :]
    a3, c3 = _bn_coefs(st3_ref, PB_ref[1:2, :], PB_ref[2:3, :], 128)
    y3 = jnp.maximum(y3pre * a3 + c3, 0.0)
    out_ref[...] = jnp.max(y3.reshape(GROUPS, K, 128), axis=1)


def _whole(shape):
    nd = len(shape)
    return pl.BlockSpec(shape, lambda i: (0,) * nd)


def _x0_spec():
    return pl.BlockSpec((TILE, 16), lambda i: (i, 0))


def _cent_spec():
    return pl.BlockSpec((GROUPS, 3), lambda i: (i, 0))


_ST = jax.ShapeDtypeStruct((8, 128), jnp.float32)
_ST_SPEC = pl.BlockSpec((8, 128), lambda i: (0, 0))


def _run_m1(x0, cent, W1p, PA):
    return pl.pallas_call(
        _m1_body, grid=(GRID_M,),
        in_specs=[_x0_spec(), _cent_spec(), _whole((64, 16)), _whole((8, 128))],
        out_specs=_ST_SPEC, out_shape=_ST,
    )(x0, cent, W1p, PA)


def _run_m2(x0, cent, W1p, W2, PA, st1):
    return pl.pallas_call(
        _m2_body, grid=(GRID_M,),
        in_specs=[_x0_spec(), _cent_spec(), _whole((64, 16)), _whole((64, 64)),
                  _whole((8, 128)), _whole((8, 128))],
        out_specs=_ST_SPEC, out_shape=_ST,
    )(x0, cent, W1p, W2, PA, st1)


def _run_m3(x0, cent, W1p, W2, W3, PA, PB, st1, st2):
    return pl.pallas_call(
        _m3_body, grid=(GRID_M,),
        in_specs=[_x0_spec(), _cent_spec(), _whole((64, 16)), _whole((64, 64)),
                  _whole((128, 64)), _whole((8, 128)), _whole((8, 128)),
                  _whole((8, 128)), _whole((8, 128))],
        out_specs=_ST_SPEC, out_shape=_ST,
    )(x0, cent, W1p, W2, W3, PA, PB, st1, st2)


def _run_m4(x0, cent, W1p, W2, W3, PA, PB, st1, st2, st3):
    return pl.pallas_call(
        _m4_body, grid=(GRID_M,),
        in_specs=[_x0_spec(), _cent_spec(), _whole((64, 16)), _whole((64, 64)),
                  _whole((128, 64)), _whole((8, 128)), _whole((8, 128)),
                  _whole((8, 128)), _whole((8, 128)), _whole((8, 128))],
        out_specs=pl.BlockSpec((GROUPS, 128), lambda i: (i, 0)),
        out_shape=jax.ShapeDtypeStruct((B * S, 128), jnp.float32),
    )(x0, cent, W1p, W2, W3, PA, PB, st1, st2, st3)


@jax.jit
def kernel(xyz, points, W1, b1, g1, bt1, W2, b2, g2, bt2, W3, b3, g3, bt3):
    xyzT = jnp.transpose(xyz, (0, 2, 1))
    xyzT3 = jnp.transpose(xyz, (2, 0, 1))
    far0 = jax.random.randint(jax.random.key(42), (B,), 0, N)
    far0 = jnp.broadcast_to(far0.astype(jnp.int32)[:, None], (B, 128))

    cents = _run_fps(xyzT3, far0)
    new_xyz = jnp.transpose(cents, (2, 0, 1))
    new_xyz_flat = new_xyz.reshape(B * S, 3)

    gidx = _run_topk(new_xyz_flat, xyzT)
    idx_sc = gidx.reshape(NW, CHUNKS, 128)

    table = jnp.concatenate(
        [xyz, points, jnp.zeros((B, N, 7), jnp.float32)], axis=-1
    ).reshape(B * N, 16)

    x0 = _sc_gather(table, idx_sc).reshape(M, 16)

    W1p = jnp.concatenate([W1, jnp.zeros((64, 7), jnp.float32)], axis=1)
    pad64 = lambda v: jnp.concatenate([v, jnp.zeros((64,), jnp.float32)])
    PA = jnp.stack([pad64(b1), pad64(g1), pad64(bt1),
                    pad64(b2), pad64(g2), pad64(bt2),
                    jnp.zeros((128,), jnp.float32),
                    jnp.zeros((128,), jnp.float32)])
    PB = jnp.stack([b3, g3, bt3] + [jnp.zeros((128,), jnp.float32)] * 5)

    st1 = _run_m1(x0, new_xyz_flat, W1p, PA)
    st2 = _run_m2(x0, new_xyz_flat, W1p, W2, PA, st1)
    st3 = _run_m3(x0, new_xyz_flat, W1p, W2, W3, PA, PB, st1, st2)
    pooled = _run_m4(x0, new_xyz_flat, W1p, W2, W3, PA, PB, st1, st2, st3)

    return new_xyz, pooled.reshape(B, S, 128)
